# Initial kernel scaffold; baseline (speedup 1.0000x reference)
import jax
import jax.numpy as jnp
from jax.experimental import pallas as pl


def kernel(intent_ids, table):
    raise NotImplementedError("write your pallas kernel here")


# speedup vs baseline: 5.0792x; 5.0792x over previous
import functools

import jax
import jax.numpy as jnp
from jax import lax
from jax.experimental import pallas as pl
from jax.experimental.pallas import tpu as pltpu
from jax.experimental.pallas import tpu_sc as plsc

_INFO = plsc.get_sparse_core_info()
_NC = _INFO.num_cores
_NS = _INFO.num_subcores
_NW = _NC * _NS

_CB = 128
_K = 5
_C = _K * _CB


@functools.lru_cache(maxsize=None)
def _make_gather(B, V, D):
    assert B % (_NW * _C) == 0
    per_w = B // _NW
    n_chunks = per_w // _C
    mesh = plsc.VectorSubcoreMesh(core_axis_name="c", subcore_axis_name="s")

    @functools.partial(
        pl.kernel,
        mesh=mesh,
        out_type=jax.ShapeDtypeStruct((B, D), jnp.float32),
        compiler_params=pltpu.CompilerParams(use_tc_tiling_on_sc=False),
        scratch_types=[
            pltpu.VMEM((2, _C), jnp.int32),
            pltpu.VMEM((2, _C, D), jnp.float32),
            pltpu.SemaphoreType.DMA,
        ],
    )
    def gather_kernel(idx_hbm, table_hbm, out_hbm, idx_v, rows_v, gsem):
        wid = lax.axis_index("s") * _NC + lax.axis_index("c")
        base = wid * per_w

        def stage_and_fire(g, buf):
            pltpu.sync_copy(idx_hbm.at[pl.ds(base + g * _C, _C)], idx_v.at[buf])
            for j in range(_K):
                pltpu.async_copy(
                    table_hbm.at[idx_v.at[buf].at[pl.ds(j * _CB, _CB)]],
                    rows_v.at[buf].at[pl.ds(j * _CB, _CB)],
                    gsem,
                )

        stage_and_fire(0, 0)

        def body(g, carry):
            buf = lax.rem(g, 2)
            nbuf = lax.rem(g + 1, 2)

            @pl.when(g + 1 < n_chunks)
            def _():
                stage_and_fire(g + 1, nbuf)

            for j in range(_K):
                pltpu.make_async_copy(
                    table_hbm.at[idx_v.at[buf].at[pl.ds(j * _CB, _CB)]],
                    rows_v.at[buf].at[pl.ds(j * _CB, _CB)],
                    gsem,
                ).wait()

            pltpu.sync_copy(rows_v.at[buf], out_hbm.at[pl.ds(base + g * _C, _C)])
            return carry

        lax.fori_loop(0, n_chunks, body, 0)

    return gather_kernel


def kernel(intent_ids, table):
    Bt, S = intent_ids.shape
    V, D = table.shape
    B = Bt * S
    idx_flat = intent_ids.reshape(B).astype(jnp.int32)
    out = _make_gather(B, V, D)(idx_flat, table)
    return out.reshape(Bt, S, D)


# speedup vs baseline: 5.1795x; 1.0197x over previous
import functools

import jax
import jax.numpy as jnp
from jax import lax
from jax.experimental import pallas as pl
from jax.experimental.pallas import tpu as pltpu
from jax.experimental.pallas import tpu_sc as plsc

_INFO = plsc.get_sparse_core_info()
_NC = _INFO.num_cores
_NS = _INFO.num_subcores
_NW = _NC * _NS

_CB = 128
_K = 5
_C = _K * _CB
_NB = 3


@functools.lru_cache(maxsize=None)
def _make_gather(B, V, D):
    assert B % (_NW * _C) == 0
    per_w = B // _NW
    n_chunks = per_w // _C
    mesh = plsc.VectorSubcoreMesh(core_axis_name="c", subcore_axis_name="s")

    @functools.partial(
        pl.kernel,
        mesh=mesh,
        out_type=jax.ShapeDtypeStruct((B, D), jnp.float32),
        compiler_params=pltpu.CompilerParams(use_tc_tiling_on_sc=False),
        scratch_types=[
            pltpu.VMEM((_NB, _C), jnp.int32),
            pltpu.VMEM((_NB, _C, D), jnp.float32),
            pltpu.SemaphoreType.DMA,
            pltpu.SemaphoreType.DMA,
        ],
    )
    def gather_kernel(idx_hbm, table_hbm, out_hbm, idx_v, rows_v, gsem, osem):
        wid = lax.axis_index("s") * _NC + lax.axis_index("c")
        base = wid * per_w

        def stage_and_fire(g, buf):
            pltpu.sync_copy(idx_hbm.at[pl.ds(base + g * _C, _C)], idx_v.at[buf])
            for j in range(_K):
                pltpu.async_copy(
                    table_hbm.at[idx_v.at[buf].at[pl.ds(j * _CB, _CB)]],
                    rows_v.at[buf].at[pl.ds(j * _CB, _CB)],
                    gsem,
                )

        def drain_gathers(buf):
            for j in range(_K):
                pltpu.make_async_copy(
                    table_hbm.at[idx_v.at[buf].at[pl.ds(j * _CB, _CB)]],
                    rows_v.at[buf].at[pl.ds(j * _CB, _CB)],
                    gsem,
                ).wait()

        def out_copy(g, buf):
            return pltpu.make_async_copy(
                rows_v.at[buf], out_hbm.at[pl.ds(base + g * _C, _C)], osem
            )

        stage_and_fire(0, 0)

        def body(g, carry):
            buf = lax.rem(g, _NB)
            nbuf = lax.rem(g + 1, _NB)

            @pl.when(g >= _NB - 1)
            def _():
                out_copy(g, nbuf).wait()

            @pl.when(g + 1 < n_chunks)
            def _():
                stage_and_fire(g + 1, nbuf)

            drain_gathers(buf)
            out_copy(g, buf).start()
            return carry

        lax.fori_loop(0, n_chunks, body, 0)

        for t in range(_NB - 1):
            out_copy(n_chunks - 1 - t, lax.rem(n_chunks - 1 - t, _NB)).wait()

    return gather_kernel


def kernel(intent_ids, table):
    Bt, S = intent_ids.shape
    V, D = table.shape
    B = Bt * S
    idx_flat = intent_ids.reshape(B).astype(jnp.int32)
    out = _make_gather(B, V, D)(idx_flat, table)
    return out.reshape(Bt, S, D)


# speedup vs baseline: 11.2331x; 2.1687x over previous
import functools

import jax
import jax.numpy as jnp
from jax import lax
from jax.experimental import pallas as pl
from jax.experimental.pallas import tpu as pltpu
from jax.experimental.pallas import tpu_sc as plsc

_INFO = plsc.get_sparse_core_info()
_NC = _INFO.num_cores
_NS = _INFO.num_subcores
_NW = _NC * _NS

_BB = 128
_NB = 3


@functools.lru_cache(maxsize=None)
def _make_gather(S, B, V, D):
    NP = S // 2
    D2 = 2 * D
    per_w = B // _NW
    n_bchunks = per_w // _BB
    mesh = plsc.VectorSubcoreMesh(core_axis_name="c", subcore_axis_name="s")

    @functools.partial(
        pl.kernel,
        mesh=mesh,
        out_type=jax.ShapeDtypeStruct((NP, B, D2), jnp.float32),
        compiler_params=pltpu.CompilerParams(use_tc_tiling_on_sc=False),
        scratch_types=[
            pltpu.VMEM((S, _BB), jnp.int32),
            pltpu.VMEM((_NB, 2, _BB, D), jnp.float32),
            pltpu.SemaphoreType.DMA,
            pltpu.SemaphoreType.DMA,
        ],
    )
    def gather_kernel(ids_hbm, table_hbm, out_hbm, idx_v, rows_v, gsem, osem):
        wid = lax.axis_index("s") * _NC + lax.axis_index("c")

        def chunk(b0):
            pltpu.sync_copy(ids_hbm.at[:, pl.ds(b0, _BB)], idx_v)

            def fire(p, buf):
                for s01 in range(2):
                    pltpu.async_copy(
                        table_hbm.at[idx_v.at[2 * p + s01]],
                        rows_v.at[buf].at[s01],
                        gsem,
                    )

            def drain(p, buf):
                for s01 in range(2):
                    pltpu.make_async_copy(
                        table_hbm.at[idx_v.at[2 * p + s01]],
                        rows_v.at[buf].at[s01],
                        gsem,
                    ).wait()

            def out_copies(p, buf):
                return [
                    pltpu.make_async_copy(
                        rows_v.at[buf].at[s01],
                        out_hbm.at[p, pl.ds(b0, _BB), pl.ds(s01 * D, D)],
                        osem,
                    )
                    for s01 in range(2)
                ]

            fire(0, 0)

            def body(p, carry):
                buf = lax.rem(p, _NB)
                nbuf = lax.rem(p + 1, _NB)

                @pl.when(p >= _NB - 1)
                def _():
                    for c in out_copies(p, nbuf):
                        c.wait()

                @pl.when(p + 1 < NP)
                def _():
                    fire(p + 1, nbuf)

                drain(p, buf)
                for c in out_copies(p, buf):
                    c.start()
                return carry

            lax.fori_loop(0, NP, body, 0)
            for t in range(_NB - 1):
                for c in out_copies(NP - 1 - t, lax.rem(NP - 1 - t, _NB)):
                    c.wait()

        for j in range(n_bchunks):
            chunk(wid * per_w + j * _BB)

    return gather_kernel


@functools.lru_cache(maxsize=None)
def _make_transpose(S, B, D):
    NP = S // 2
    D2 = 2 * D

    def body(x_ref, o_ref):
        x = x_ref[...]
        y = jnp.transpose(x, (0, 2, 1))
        o_ref[...] = y.reshape(S, D, _BB)

    return pl.pallas_call(
        body,
        grid=(B // _BB,),
        in_specs=[pl.BlockSpec((NP, _BB, D2), lambda i: (0, i, 0))],
        out_specs=pl.BlockSpec((S, D, _BB), lambda i: (0, 0, i)),
        out_shape=jax.ShapeDtypeStruct((S, D, B), jnp.float32),
    )


def kernel(intent_ids, table):
    Bt, S = intent_ids.shape
    V, D = table.shape
    ids_t = jnp.transpose(intent_ids).astype(jnp.int32)
    out1 = _make_gather(S, Bt, V, D)(ids_t, table)
    out2 = _make_transpose(S, Bt, D)(out1)
    return jnp.transpose(out2, (2, 0, 1))


# speedup vs baseline: 11.3422x; 1.0097x over previous
import functools

import jax
import jax.numpy as jnp
from jax import lax
from jax.experimental import pallas as pl
from jax.experimental.pallas import tpu as pltpu
from jax.experimental.pallas import tpu_sc as plsc

_INFO = plsc.get_sparse_core_info()
_NC = _INFO.num_cores
_NS = _INFO.num_subcores
_NW = _NC * _NS

_BB = 128
_NB = 3
_SPLIT = 4


@functools.lru_cache(maxsize=None)
def _make_gather(S, B, V, D, b_base, b_len):
    NP = S // 2
    D2 = 2 * D
    per_w = b_len // _NW
    n_bchunks = per_w // _BB
    mesh = plsc.VectorSubcoreMesh(core_axis_name="c", subcore_axis_name="s")

    @functools.partial(
        pl.kernel,
        mesh=mesh,
        out_type=jax.ShapeDtypeStruct((NP, b_len, D2), jnp.float32),
        compiler_params=pltpu.CompilerParams(use_tc_tiling_on_sc=False),
        scratch_types=[
            pltpu.VMEM((S, _BB), jnp.int32),
            pltpu.VMEM((_NB, 2, _BB, D), jnp.float32),
            pltpu.SemaphoreType.DMA,
            pltpu.SemaphoreType.DMA,
        ],
    )
    def gather_kernel(ids_hbm, table_hbm, out_hbm, idx_v, rows_v, gsem, osem):
        wid = lax.axis_index("s") * _NC + lax.axis_index("c")

        def chunk(b0):
            pltpu.sync_copy(ids_hbm.at[:, pl.ds(b_base + b0, _BB)], idx_v)

            def fire(p, buf):
                for s01 in range(2):
                    pltpu.async_copy(
                        table_hbm.at[idx_v.at[2 * p + s01]],
                        rows_v.at[buf].at[s01],
                        gsem,
                    )

            def drain(p, buf):
                for s01 in range(2):
                    pltpu.make_async_copy(
                        table_hbm.at[idx_v.at[2 * p + s01]],
                        rows_v.at[buf].at[s01],
                        gsem,
                    ).wait()

            def out_copies(p, buf):
                return [
                    pltpu.make_async_copy(
                        rows_v.at[buf].at[s01],
                        out_hbm.at[p, pl.ds(b0, _BB), pl.ds(s01 * D, D)],
                        osem,
                    )
                    for s01 in range(2)
                ]

            fire(0, 0)

            def body(p, carry):
                buf = lax.rem(p, _NB)
                nbuf = lax.rem(p + 1, _NB)

                @pl.when(p >= _NB - 1)
                def _():
                    for c in out_copies(p, nbuf):
                        c.wait()

                @pl.when(p + 1 < NP)
                def _():
                    fire(p + 1, nbuf)

                drain(p, buf)
                for c in out_copies(p, buf):
                    c.start()
                return carry

            lax.fori_loop(0, NP, body, 0)
            for t in range(_NB - 1):
                for c in out_copies(NP - 1 - t, lax.rem(NP - 1 - t, _NB)):
                    c.wait()

        for j in range(n_bchunks):
            chunk(wid * per_w + j * _BB)

    return gather_kernel


@functools.lru_cache(maxsize=None)
def _make_transpose(S, B, D, b_base, b_len, first):
    NP = S // 2
    D2 = 2 * D
    blk0 = b_base // _BB

    def body(*refs):
        x_ref, o_ref = refs[-2], refs[-1]
        x = x_ref[...]
        y = jnp.transpose(x, (0, 2, 1))
        o_ref[...] = y.reshape(S, D, _BB)

    piece_spec = pl.BlockSpec((NP, _BB, D2), lambda i: (0, i, 0))
    if first:
        in_specs = [piece_spec]
        aliases = {}
    else:
        in_specs = [pl.BlockSpec(memory_space=pl.ANY), piece_spec]
        aliases = {0: 0}

    return pl.pallas_call(
        body,
        grid=(b_len // _BB,),
        in_specs=in_specs,
        out_specs=pl.BlockSpec((S, D, _BB), lambda i: (0, 0, blk0 + i)),
        out_shape=jax.ShapeDtypeStruct((S, D, B), jnp.float32),
        input_output_aliases=aliases,
    )


def kernel(intent_ids, table):
    Bt, S = intent_ids.shape
    V, D = table.shape
    ids_t = jnp.transpose(intent_ids).astype(jnp.int32)
    piece = Bt // _SPLIT
    outs1 = [
        _make_gather(S, Bt, V, D, j * piece, piece)(ids_t, table)
        for j in range(_SPLIT)
    ]
    out2 = _make_transpose(S, Bt, D, 0, piece, True)(outs1[0])
    for j in range(1, _SPLIT):
        out2 = _make_transpose(S, Bt, D, j * piece, piece, False)(out2, outs1[j])
    return jnp.transpose(out2, (2, 0, 1))
